# Initial kernel scaffold; baseline (speedup 1.0000x reference)
import jax
import jax.numpy as jnp
from jax.experimental import pallas as pl


def kernel(dec_X, table, W_h, W_out, h0):
    raise NotImplementedError("write your pallas kernel here")


# speedup vs baseline: 10.6362x; 10.6362x over previous
import jax
import jax.numpy as jnp
from jax.experimental import pallas as pl
from jax.experimental.pallas import tpu as pltpu

_BATCH = 64
_BEAM = 8
_R = _BATCH * _BEAM
_D = 64
_V = 100000
_EOS = 1
_NEG = -1e9
_STEPS = 8

_T = 2048
_NT = 49
_VP = _T * _NT
_SUB = 16
_LANES = 128
_NSEG = _NT * _LANES


def _step_kernel(emb_ref, hprev_ref, wh_ref, wout_ref, hnew_ref, segmax_ref, stats_ref):
    j = pl.program_id(0)

    @pl.when(j == 0)
    def _init():
        hn = jnp.tanh(
            jnp.dot(emb_ref[...], wh_ref[...], preferred_element_type=jnp.float32)
            + hprev_ref[...]
        )
        hnew_ref[...] = hn
        stats_ref[:, 0:1] = jnp.full((_R, 1), -1e30, jnp.float32)
        stats_ref[:, 1:2] = jnp.zeros((_R, 1), jnp.float32)

    h = hnew_ref[...]
    tile = jnp.dot(h, wout_ref[...], preferred_element_type=jnp.float32)
    col = j * _T + jax.lax.broadcasted_iota(jnp.int32, (_R, _T), 1)
    tile = jnp.where(col < _V, tile, _NEG)

    tmax = jnp.max(tile, axis=1, keepdims=True)
    mold = stats_ref[:, 0:1]
    sold = stats_ref[:, 1:2]
    mnew = jnp.maximum(mold, tmax)
    snew = sold * jnp.exp(mold - mnew) + jnp.sum(
        jnp.exp(tile - mnew), axis=1, keepdims=True
    )
    stats_ref[:, 0:1] = mnew
    stats_ref[:, 1:2] = snew

    seg = tile[:, 0:_LANES]
    for i in range(1, _SUB):
        seg = jnp.maximum(seg, tile[:, i * _LANES:(i + 1) * _LANES])
    segmax_ref[...] = seg


def _decode_step(emb, hprev, wh, wout_p):
    return pl.pallas_call(
        _step_kernel,
        grid=(_NT,),
        in_specs=[
            pl.BlockSpec((_R, _D), lambda j: (0, 0)),
            pl.BlockSpec((_R, _D), lambda j: (0, 0)),
            pl.BlockSpec((_D, _D), lambda j: (0, 0)),
            pl.BlockSpec((_D, _T), lambda j: (0, j)),
        ],
        out_specs=[
            pl.BlockSpec((_R, _D), lambda j: (0, 0)),
            pl.BlockSpec((_R, _LANES), lambda j: (0, j)),
            pl.BlockSpec((_R, 128), lambda j: (0, 0)),
        ],
        out_shape=[
            jax.ShapeDtypeStruct((_R, _D), jnp.float32),
            jax.ShapeDtypeStruct((_R, _NSEG), jnp.float32),
            jax.ShapeDtypeStruct((_R, 128), jnp.float32),
        ],
        compiler_params=pltpu.CompilerParams(dimension_semantics=("arbitrary",)),
    )(emb, hprev, wh, wout_p)


def kernel(dec_X, table, W_h, W_out, h0):
    wout_p = jnp.pad(W_out, ((0, 0), (0, _VP - _V)))
    wt = W_out.T
    h = jnp.repeat(h0, _BEAM, axis=0)
    x = jnp.repeat(dec_X[:, 0], _BEAM)
    scores = jnp.full((_BATCH, _BEAM), _NEG, jnp.float32).at[:, 0].set(0.0)
    finished = jnp.zeros((_BATCH, _BEAM), dtype=bool)
    seqs = None
    for _ in range(_STEPS):
        emb = table[x]
        hnew, segmax, stats = _decode_step(emb, h, W_h, wout_p)
        lse = stats[:, 0:1] + jnp.log(stats[:, 1:2])

        _, seg = jax.lax.top_k(segmax, _BEAM)
        tj = seg // _LANES
        lane = seg % _LANES
        cols = (tj * _T + lane)[:, :, None] + _LANES * jnp.arange(_SUB)[None, None, :]
        cols = cols.reshape(_R, _BEAM * _SUB).astype(jnp.int32)
        wc = wt[jnp.clip(cols, 0, _V - 1)]
        cl = jnp.einsum("rkd,rd->rk", wc, hnew)
        logp = cl - lse

        fin_flat = finished.reshape(_R)
        sc_flat = scores.reshape(_R)
        cand = sc_flat[:, None] + logp
        cand = jnp.where((cols < _V) & (~fin_flat[:, None]), cand, _NEG)
        extra = jnp.where(fin_flat, sc_flat, _NEG)[:, None]
        cand_all = jnp.concatenate([cand, extra], axis=1)
        tok_all = jnp.concatenate(
            [cols, jnp.full((_R, 1), _EOS, jnp.int32)], axis=1
        )

        bc = cand_all.reshape(_BATCH, _BEAM * 129)
        bt = tok_all.reshape(_BATCH, _BEAM * 129)
        topv, topi = jax.lax.top_k(bc, _BEAM)
        beam_idx = topi // 129
        tok = jnp.take_along_axis(bt, topi, axis=1)

        if seqs is None:
            seqs = tok[:, :, None]
        else:
            g = jnp.broadcast_to(beam_idx[:, :, None], (_BATCH, _BEAM, seqs.shape[2]))
            seqs = jnp.take_along_axis(seqs, g, axis=1)
            seqs = jnp.concatenate([seqs, tok[:, :, None]], axis=2)
        finished = jnp.take_along_axis(finished, beam_idx, axis=1) | (tok == _EOS)
        flat = ((jnp.arange(_BATCH) * _BEAM)[:, None] + beam_idx).reshape(-1)
        h = hnew[flat]
        x = tok.reshape(-1)
        scores = topv
    best = jnp.argmax(scores, axis=1)
    pred = seqs[jnp.arange(_BATCH), best]
    return pred, scores


# speedup vs baseline: 15.5621x; 1.4631x over previous
import jax
import jax.numpy as jnp
from jax.experimental import pallas as pl
from jax.experimental.pallas import tpu as pltpu

_BATCH = 64
_BEAM = 8
_R = _BATCH * _BEAM
_D = 64
_V = 100000
_EOS = 1
_NEG = -1e9
_STEPS = 8

_T = 2048
_NT = 49
_VP = _T * _NT
_SUB = 16
_LANES = 128
_NSEG = _NT * _LANES


def _step_kernel(emb_ref, hprev_ref, wh_ref, wout_ref,
                 hnew_ref, stats_ref, segid_ref, segmax_scr):
    j = pl.program_id(0)

    @pl.when(j == 0)
    def _init():
        hn = jnp.tanh(
            jnp.dot(emb_ref[...], wh_ref[...], preferred_element_type=jnp.float32)
            + hprev_ref[...]
        )
        hnew_ref[...] = hn
        stats_ref[:, 0:1] = jnp.full((_R, 1), -1e30, jnp.float32)
        stats_ref[:, 1:2] = jnp.zeros((_R, 1), jnp.float32)

    h = hnew_ref[...]
    tile = jnp.dot(h, wout_ref[...], preferred_element_type=jnp.float32)
    col = j * _T + jax.lax.broadcasted_iota(jnp.int32, (_R, _T), 1)
    tile = jnp.where(col < _V, tile, _NEG)

    tmax = jnp.max(tile, axis=1, keepdims=True)
    mold = stats_ref[:, 0:1]
    sold = stats_ref[:, 1:2]
    mnew = jnp.maximum(mold, tmax)
    snew = sold * jnp.exp(mold - mnew) + jnp.sum(
        jnp.exp(tile - mnew), axis=1, keepdims=True
    )
    stats_ref[:, 0:1] = mnew
    stats_ref[:, 1:2] = snew

    seg = tile[:, 0:_LANES]
    for i in range(1, _SUB):
        seg = jnp.maximum(seg, tile[:, i * _LANES:(i + 1) * _LANES])
    segmax_scr[:, pl.ds(j * _LANES, _LANES)] = seg

    @pl.when(j == _NT - 1)
    def _select():
        s = segmax_scr[...]
        lane = jax.lax.broadcasted_iota(jnp.int32, (_R, _NSEG), 1)
        for k in range(_BEAM):
            m = jnp.max(s, axis=1, keepdims=True)
            pos = jnp.min(
                jnp.where(s == m, lane, jnp.int32(2 ** 30)),
                axis=1, keepdims=True,
            )
            segid_ref[:, k:k + 1] = pos
            s = jnp.where(lane == pos, _NEG * 2.0, s)


def _decode_step(emb, hprev, wh, wout_p):
    return pl.pallas_call(
        _step_kernel,
        grid=(_NT,),
        in_specs=[
            pl.BlockSpec((_R, _D), lambda j: (0, 0)),
            pl.BlockSpec((_R, _D), lambda j: (0, 0)),
            pl.BlockSpec((_D, _D), lambda j: (0, 0)),
            pl.BlockSpec((_D, _T), lambda j: (0, j)),
        ],
        out_specs=[
            pl.BlockSpec((_R, _D), lambda j: (0, 0)),
            pl.BlockSpec((_R, 128), lambda j: (0, 0)),
            pl.BlockSpec((_R, _BEAM), lambda j: (0, 0)),
        ],
        out_shape=[
            jax.ShapeDtypeStruct((_R, _D), jnp.float32),
            jax.ShapeDtypeStruct((_R, 128), jnp.float32),
            jax.ShapeDtypeStruct((_R, _BEAM), jnp.int32),
        ],
        scratch_shapes=[pltpu.VMEM((_R, _NSEG), jnp.float32)],
        compiler_params=pltpu.CompilerParams(dimension_semantics=("arbitrary",)),
    )(emb, hprev, wh, wout_p)


def kernel(dec_X, table, W_h, W_out, h0):
    wout_p = jnp.pad(W_out, ((0, 0), (0, _VP - _V)))
    wt = W_out.T
    h = jnp.repeat(h0, _BEAM, axis=0)
    x = jnp.repeat(dec_X[:, 0], _BEAM)
    scores = jnp.full((_BATCH, _BEAM), _NEG, jnp.float32).at[:, 0].set(0.0)
    finished = jnp.zeros((_BATCH, _BEAM), dtype=bool)
    seqs = None
    for _ in range(_STEPS):
        emb = table[x]
        hnew, stats, seg = _decode_step(emb, h, W_h, wout_p)
        lse = stats[:, 0:1] + jnp.log(stats[:, 1:2])

        tj = seg // _LANES
        lane = seg % _LANES
        cols = (tj * _T + lane)[:, :, None] + _LANES * jnp.arange(_SUB)[None, None, :]
        cols = cols.reshape(_R, _BEAM * _SUB).astype(jnp.int32)
        wc = wt[jnp.clip(cols, 0, _V - 1)]
        cl = jnp.einsum("rkd,rd->rk", wc, hnew)
        logp = cl - lse

        fin_flat = finished.reshape(_R)
        sc_flat = scores.reshape(_R)
        cand = sc_flat[:, None] + logp
        cand = jnp.where((cols < _V) & (~fin_flat[:, None]), cand, _NEG)
        extra = jnp.where(fin_flat, sc_flat, _NEG)[:, None]
        cand_all = jnp.concatenate([cand, extra], axis=1)
        tok_all = jnp.concatenate(
            [cols, jnp.full((_R, 1), _EOS, jnp.int32)], axis=1
        )

        bc = cand_all.reshape(_BATCH, _BEAM * 129)
        bt = tok_all.reshape(_BATCH, _BEAM * 129)
        topv, topi = jax.lax.top_k(bc, _BEAM)
        beam_idx = topi // 129
        tok = jnp.take_along_axis(bt, topi, axis=1)

        if seqs is None:
            seqs = tok[:, :, None]
        else:
            g = jnp.broadcast_to(beam_idx[:, :, None], (_BATCH, _BEAM, seqs.shape[2]))
            seqs = jnp.take_along_axis(seqs, g, axis=1)
            seqs = jnp.concatenate([seqs, tok[:, :, None]], axis=2)
        finished = jnp.take_along_axis(finished, beam_idx, axis=1) | (tok == _EOS)
        flat = ((jnp.arange(_BATCH) * _BEAM)[:, None] + beam_idx).reshape(-1)
        h = hnew[flat]
        x = tok.reshape(-1)
        scores = topv
    best = jnp.argmax(scores, axis=1)
    pred = seqs[jnp.arange(_BATCH), best]
    return pred, scores


# speedup vs baseline: 28.7148x; 1.8452x over previous
import functools

import jax
import jax.numpy as jnp
from jax import lax
from jax.experimental import pallas as pl
from jax.experimental.pallas import tpu as pltpu
from jax.experimental.pallas import tpu_sc as plsc

_BATCH = 64
_BEAM = 8
_R = _BATCH * _BEAM
_D = 64
_V = 100000
_EOS = 1
_NEG = -1e9
_STEPS = 8

_T = 2048
_NT = 49
_VP = _T * _NT
_SUB = 16
_LANES = 128
_NSEG = _NT * _LANES


def _step_kernel(emb_ref, hprev_ref, wh_ref, wout_ref,
                 hnew_ref, stats_ref, segid_ref, segmax_scr):
    j = pl.program_id(0)

    @pl.when(j == 0)
    def _init():
        hn = jnp.tanh(
            jnp.dot(emb_ref[...], wh_ref[...], preferred_element_type=jnp.float32)
            + hprev_ref[...]
        )
        hnew_ref[...] = hn
        stats_ref[:, 0:1] = jnp.full((_R, 1), -1e30, jnp.float32)
        stats_ref[:, 1:2] = jnp.zeros((_R, 1), jnp.float32)

    h = hnew_ref[...]
    tile = jnp.dot(h, wout_ref[...], preferred_element_type=jnp.float32)
    col = j * _T + jax.lax.broadcasted_iota(jnp.int32, (_R, _T), 1)
    tile = jnp.where(col < _V, tile, _NEG)

    tmax = jnp.max(tile, axis=1, keepdims=True)
    mold = stats_ref[:, 0:1]
    sold = stats_ref[:, 1:2]
    mnew = jnp.maximum(mold, tmax)
    snew = sold * jnp.exp(mold - mnew) + jnp.sum(
        jnp.exp(tile - mnew), axis=1, keepdims=True
    )
    stats_ref[:, 0:1] = mnew
    stats_ref[:, 1:2] = snew

    seg = tile[:, 0:_LANES]
    for i in range(1, _SUB):
        seg = jnp.maximum(seg, tile[:, i * _LANES:(i + 1) * _LANES])
    segmax_scr[:, pl.ds(j * _LANES, _LANES)] = seg

    @pl.when(j == _NT - 1)
    def _select():
        s = segmax_scr[...]
        lane = jax.lax.broadcasted_iota(jnp.int32, (_R, _NSEG), 1)
        for k in range(_BEAM):
            m = jnp.max(s, axis=1, keepdims=True)
            pos = jnp.min(
                jnp.where(s == m, lane, jnp.int32(2 ** 30)),
                axis=1, keepdims=True,
            )
            segid_ref[:, k:k + 1] = pos
            s = jnp.where(lane == pos, _NEG * 2.0, s)


_NC = 2
_NS = 16
_NW = _NC * _NS
_GB = _R * _BEAM * _SUB
_BPW = _GB // _NW
_CH = 1024


def _sc_gather_body(wt_hbm, idx_hbm, out_hbm, idx_v, rows_v, sem):
    wid = lax.axis_index("s") * _NC + lax.axis_index("c")
    base = wid * _BPW
    for c in range(_BPW // _CH):
        off = base + c * _CH
        pltpu.sync_copy(idx_hbm.at[pl.ds(off, _CH)], idx_v)
        pltpu.async_copy(wt_hbm.at[idx_v], rows_v, sem).wait()
        pltpu.sync_copy(rows_v, out_hbm.at[pl.ds(off, _CH)])


_sc_gather = functools.partial(
    pl.kernel,
    mesh=plsc.VectorSubcoreMesh(core_axis_name="c", subcore_axis_name="s"),
    out_type=jax.ShapeDtypeStruct((_GB, _D), jnp.float32),
    scratch_types=[
        pltpu.VMEM((_CH,), jnp.int32),
        pltpu.VMEM((_CH, _D), jnp.float32),
        pltpu.SemaphoreType.DMA,
    ],
    compiler_params=pltpu.CompilerParams(use_tc_tiling_on_sc=False),
)(_sc_gather_body)


def _decode_step(emb, hprev, wh, wout_p):
    return pl.pallas_call(
        _step_kernel,
        grid=(_NT,),
        in_specs=[
            pl.BlockSpec((_R, _D), lambda j: (0, 0)),
            pl.BlockSpec((_R, _D), lambda j: (0, 0)),
            pl.BlockSpec((_D, _D), lambda j: (0, 0)),
            pl.BlockSpec((_D, _T), lambda j: (0, j)),
        ],
        out_specs=[
            pl.BlockSpec((_R, _D), lambda j: (0, 0)),
            pl.BlockSpec((_R, 128), lambda j: (0, 0)),
            pl.BlockSpec((_R, _BEAM), lambda j: (0, 0)),
        ],
        out_shape=[
            jax.ShapeDtypeStruct((_R, _D), jnp.float32),
            jax.ShapeDtypeStruct((_R, 128), jnp.float32),
            jax.ShapeDtypeStruct((_R, _BEAM), jnp.int32),
        ],
        scratch_shapes=[pltpu.VMEM((_R, _NSEG), jnp.float32)],
        compiler_params=pltpu.CompilerParams(dimension_semantics=("arbitrary",)),
    )(emb, hprev, wh, wout_p)


def kernel(dec_X, table, W_h, W_out, h0):
    wout_p = jnp.pad(W_out, ((0, 0), (0, _VP - _V)))
    wt = W_out.T
    h = jnp.repeat(h0, _BEAM, axis=0)
    x = jnp.repeat(dec_X[:, 0], _BEAM)
    scores = jnp.full((_BATCH, _BEAM), _NEG, jnp.float32).at[:, 0].set(0.0)
    finished = jnp.zeros((_BATCH, _BEAM), dtype=bool)
    seqs = None
    for _ in range(_STEPS):
        emb = table[x]
        hnew, stats, seg = _decode_step(emb, h, W_h, wout_p)
        lse = stats[:, 0:1] + jnp.log(stats[:, 1:2])

        tj = seg // _LANES
        lane = seg % _LANES
        cols = (tj * _T + lane)[:, :, None] + _LANES * jnp.arange(_SUB)[None, None, :]
        cols = cols.reshape(_R, _BEAM * _SUB).astype(jnp.int32)
        flat_cols = jnp.clip(cols, 0, _V - 1).reshape(_GB)
        wc = _sc_gather(wt, flat_cols).reshape(_R, _BEAM * _SUB, _D)
        cl = jnp.einsum("rkd,rd->rk", wc, hnew)
        logp = cl - lse

        fin_flat = finished.reshape(_R)
        sc_flat = scores.reshape(_R)
        cand = sc_flat[:, None] + logp
        cand = jnp.where((cols < _V) & (~fin_flat[:, None]), cand, _NEG)
        extra = jnp.where(fin_flat, sc_flat, _NEG)[:, None]
        cand_all = jnp.concatenate([cand, extra], axis=1)
        tok_all = jnp.concatenate(
            [cols, jnp.full((_R, 1), _EOS, jnp.int32)], axis=1
        )

        bc = cand_all.reshape(_BATCH, _BEAM * 129)
        bt = tok_all.reshape(_BATCH, _BEAM * 129)
        topv, topi = jax.lax.top_k(bc, _BEAM)
        beam_idx = topi // 129
        tok = jnp.take_along_axis(bt, topi, axis=1)

        if seqs is None:
            seqs = tok[:, :, None]
        else:
            g = jnp.broadcast_to(beam_idx[:, :, None], (_BATCH, _BEAM, seqs.shape[2]))
            seqs = jnp.take_along_axis(seqs, g, axis=1)
            seqs = jnp.concatenate([seqs, tok[:, :, None]], axis=2)
        finished = jnp.take_along_axis(finished, beam_idx, axis=1) | (tok == _EOS)
        flat = ((jnp.arange(_BATCH) * _BEAM)[:, None] + beam_idx).reshape(-1)
        h = hnew[flat]
        x = tok.reshape(-1)
        scores = topv
    best = jnp.argmax(scores, axis=1)
    pred = seqs[jnp.arange(_BATCH), best]
    return pred, scores
